# baseline (speedup 1.0000x reference)
import functools

import jax
import jax.numpy as jnp
from jax import lax
from jax.experimental import pallas as pl
from jax.experimental.pallas import tpu as pltpu
from jax.experimental.pallas import tpu_sc as plsc

NQ = 1024
NKEYS = 100000
CHUNK = 128
NCHUNK = 784
NPAD = NCHUNK * CHUNK
KB = 2048
TOPK = 100
NSEL = TOPK * CHUNK
NEG = -3.0e38
BIGI = 2**30

def _mm_body(q_ref, k_ref, out_ref, m_ref):
    j = pl.program_id(0)
    s = lax.dot_general(q_ref[...], k_ref[...],
                        dimension_numbers=(((1,), (1,)), ((), ())),
                        preferred_element_type=jnp.float32)
    col = j * KB + lax.broadcasted_iota(jnp.int32, s.shape, 1)
    s = jnp.where(col < NKEYS, s, NEG)
    out_ref[...] = s
    m = jnp.max(s.reshape(NQ, KB // CHUNK, CHUNK), axis=2)
    m_ref[...] = m.reshape(1, NQ, KB // CHUNK)


def _matmul(queries, keys_p):
    return pl.pallas_call(
        _mm_body,
        grid=(NPAD // KB,),
        in_specs=[
            pl.BlockSpec((NQ, 128), lambda j: (0, 0)),
            pl.BlockSpec((KB, 128), lambda j: (j, 0)),
        ],
        out_specs=[
            pl.BlockSpec((NQ, KB), lambda j: (0, j)),
            pl.BlockSpec((1, NQ, KB // CHUNK), lambda j: (j, 0, 0)),
        ],
        out_shape=[
            jax.ShapeDtypeStruct((NQ, NPAD), jnp.float32),
            jax.ShapeDtypeStruct((NPAD // KB, NQ, KB // CHUNK), jnp.float32),
        ],
    )(queries, keys_p)


MASK31 = 0x7FFFFFFF


def _tau_descent(vals, sbuf, rows):
    ib = lax.bitcast_convert_type(vals, jnp.int32)
    sbuf[...] = jnp.where(ib >= 0, ib, ib ^ MASK31)
    minint = jnp.int32(-2**31)

    def body(i, prefix_v):
        trial_v = prefix_v | lax.shift_left(jnp.int32(1), 31 - i)
        trial_s = trial_v ^ minint
        cnt = jnp.sum((sbuf[...] >= trial_s[:, None]).astype(jnp.int32),
                      axis=1)
        return jnp.where(cnt >= TOPK, trial_v, prefix_v)

    pfx_v = lax.fori_loop(0, 32, body, jnp.zeros((rows,), jnp.int32))
    pfx = pfx_v ^ minint
    tb = jnp.where(pfx >= 0, pfx, pfx ^ MASK31)
    return lax.bitcast_convert_type(tb, jnp.float32).reshape(rows, 1)


def _tau_body(w, m_ref, tau_ref, sbuf):
    tau_ref[...] = _tau_descent(m_ref[...], sbuf, NQ)


def _tau(m, w):
    return pl.pallas_call(
        functools.partial(_tau_body, w),
        out_shape=jax.ShapeDtypeStruct((NQ, 1), jnp.float32),
        scratch_shapes=[pltpu.VMEM((NQ, w), jnp.int32)],
    )(m)


CB = 8
OUTW = 144


def _compact_body(w, m_hbm, tau_hbm, ids_hbm, row_v, tau_v, out_v):
    wid = lax.axis_index("s") * 2 + lax.axis_index("c")
    base = wid * SUBQ
    pltpu.sync_copy(tau_hbm.at[pl.ds(base, SUBQ)], tau_v)
    iota16 = lax.broadcasted_iota(jnp.int32, (16,), 0)

    def per_b(bi, _):
        qb = bi * CB
        pltpu.sync_copy(m_hbm.at[pl.ds(base + qb, CB)], row_v)

        def per_q(l, _1):
            tsp = plsc.load_gather(tau_v, [jnp.full((16,), qb + l, jnp.int32)])

            def per_g(g, cur):
                v = row_v[l, pl.ds(g * 16, 16)]
                msk = v >= tsp
                mi = msk.astype(jnp.int32)
                excl = plsc.cumsum(mi) - mi
                pos = jnp.minimum(excl + cur, OUTW - 1)
                ids = plsc.bitcast(g * 16 + iota16, jnp.float32)
                plsc.store_scatter(out_v, [jnp.full((16,), l, jnp.int32),
                                           pos], ids, mask=msk)
                return cur + jnp.sum(mi)

            lax.fori_loop(0, w // 16, per_g, 0)
            return 0

        lax.fori_loop(0, CB, per_q, 0)
        pltpu.sync_copy(out_v.at[:, pl.ds(0, 128)],
                        ids_hbm.at[pl.ds(base + qb, CB)])
        return 0

    lax.fori_loop(0, SUBQ // CB, per_b, 0)


def _compact(m, tau_flat, w):
    mesh = plsc.VectorSubcoreMesh(core_axis_name="c", subcore_axis_name="s")
    f = functools.partial(
        pl.kernel,
        out_type=jax.ShapeDtypeStruct((NQ, 128), jnp.float32),
        mesh=mesh,
        scratch_types=[
            pltpu.VMEM((CB, w), jnp.float32),
            pltpu.VMEM((SUBQ,), jnp.float32),
            pltpu.VMEM((CB, OUTW), jnp.float32),
        ],
        compiler_params=pltpu.CompilerParams(needs_layout_passes=False),
    )(functools.partial(_compact_body, w))
    return lax.bitcast_convert_type(f(m, tau_flat), jnp.int32)


def _select(m, w):
    tau = _tau(m, w)
    return _compact(m, tau.reshape(NQ), w)


NW = 32
ROWS_TOTAL = NQ * TOPK
ROWS_PER_W = ROWS_TOTAL // NW
WIN = 128
NWIN = ROWS_PER_W // WIN


def _gather_body(width, table_hbm, idx_hbm, out_hbm, idx_v, rows_v, sem):
    wid = lax.axis_index("s") * 2 + lax.axis_index("c")
    base0 = wid * ROWS_PER_W

    def win(w, _):
        base = base0 + w * WIN
        pltpu.sync_copy(idx_hbm.at[pl.ds(base, WIN)], idx_v)
        pltpu.async_copy(table_hbm.at[idx_v], rows_v, sem).wait()
        pltpu.sync_copy(rows_v, out_hbm.at[pl.ds(base, WIN)])
        return 0

    lax.fori_loop(0, NWIN, win, 0)


def _gather(table, idx_flat, width):
    mesh = plsc.VectorSubcoreMesh(core_axis_name="c", subcore_axis_name="s")
    f = functools.partial(
        pl.kernel,
        out_type=jax.ShapeDtypeStruct((ROWS_TOTAL, width), jnp.float32),
        mesh=mesh,
        scratch_types=[
            pltpu.VMEM((WIN,), jnp.int32),
            pltpu.VMEM((WIN, width), jnp.float32),
            pltpu.SemaphoreType.DMA,
        ],
    )(functools.partial(_gather_body, width))
    return f(table, idx_flat)


SUBQ = NQ // NW


SB = 4


def _subgather_body(cand_hbm, sel2_hbm, selrow_hbm, vals_hbm, docs_hbm,
                    row_v, ids_v, selrow_v, out_v, doc_v):
    wid = lax.axis_index("s") * 2 + lax.axis_index("c")
    base = wid * SUBQ
    iota16 = lax.broadcasted_iota(jnp.int32, (16,), 0)

    def per_b(bi, _):
        qb = base + bi * SB
        pltpu.sync_copy(cand_hbm.at[pl.ds(qb, SB)], row_v)
        pltpu.sync_copy(sel2_hbm.at[pl.ds(qb, SB)], ids_v)
        pltpu.sync_copy(selrow_hbm.at[pl.ds(qb, SB)], selrow_v)

        def per_q(l, _1):
            lsplat = jnp.full((16,), l, jnp.int32)

            def per_j(j, _2):
                jsplat = jnp.full((16,), j, dtype=jnp.int32)
                sid_f = plsc.load_gather(ids_v, [lsplat, jsplat])
                sid = plsc.bitcast(sid_f, jnp.int32)
                pos = sid + NSUB * iota16
                out_v[l, pl.ds(j * SUB, SUB)] = (
                    plsc.load_gather(row_v, [lsplat, pos]))
                crank = lax.shift_right_logical(pos, 7)
                cid = plsc.bitcast(
                    plsc.load_gather(selrow_v, [lsplat, crank]), jnp.int32)
                doc_v[l, pl.ds(j * SUB, SUB)] = (cid * CHUNK
                                                 + (pos & (CHUNK - 1)))
                return 0

            lax.fori_loop(0, TOPK, per_j, 0)
            return 0

        lax.fori_loop(0, SB, per_q, 0)
        pltpu.sync_copy(out_v, vals_hbm.at[pl.ds(qb, SB)])
        pltpu.sync_copy(doc_v, docs_hbm.at[pl.ds(qb, SB)])
        return 0

    lax.fori_loop(0, SUBQ // SB, per_b, 0)


def _subgather(cand, sel2_rows, sel_rows):
    mesh = plsc.VectorSubcoreMesh(core_axis_name="c", subcore_axis_name="s")
    f = functools.partial(
        pl.kernel,
        out_type=[
            jax.ShapeDtypeStruct((NQ, TOPK * SUB), jnp.float32),
            jax.ShapeDtypeStruct((NQ, TOPK * SUB), jnp.int32),
        ],
        mesh=mesh,
        scratch_types=[
            pltpu.VMEM((SB, NSEL), jnp.float32),
            pltpu.VMEM((SB, 128), jnp.float32),
            pltpu.VMEM((SB, 128), jnp.float32),
            pltpu.VMEM((SB, TOPK * SUB), jnp.float32),
            pltpu.VMEM((SB, TOPK * SUB), jnp.int32),
        ],
        compiler_params=pltpu.CompilerParams(needs_layout_passes=False),
    )(_subgather_body)
    return f(cand, lax.bitcast_convert_type(sel2_rows, jnp.float32),
             lax.bitcast_convert_type(sel_rows, jnp.float32))


SUB = 16
NSUB = NSEL // SUB
QBA = 256


def _submax_body(c_ref, m2_ref, tau_ref, sbuf):
    u = c_ref[...]
    h = NSEL
    while h > NSUB:
        h //= 2
        u = jnp.maximum(u[:, :h], u[:, h:])
    m2_ref[...] = u
    tau_ref[...] = _tau_descent(u, sbuf, QBA)


def _submax(cand):
    return pl.pallas_call(
        _submax_body,
        grid=(NQ // QBA,),
        in_specs=[pl.BlockSpec((QBA, NSEL), lambda qi: (qi, 0))],
        out_specs=[
            pl.BlockSpec((QBA, NSUB), lambda qi: (qi, 0)),
            pl.BlockSpec((QBA, 1), lambda qi: (qi, 0)),
        ],
        out_shape=[
            jax.ShapeDtypeStruct((NQ, NSUB), jnp.float32),
            jax.ShapeDtypeStruct((NQ, 1), jnp.float32),
        ],
        scratch_shapes=[pltpu.VMEM((QBA, NSUB), jnp.int32)],
    )(cand)


NFIN = TOPK * SUB


def _cpair_body(w, v_hbm, d_hbm, tau_hbm, vo_hbm, do_hbm,
                row_v, drow_v, tau_v, vout_v, dout_v):
    wid = lax.axis_index("s") * 2 + lax.axis_index("c")
    base = wid * SUBQ
    pltpu.sync_copy(tau_hbm.at[pl.ds(base, SUBQ)], tau_v)
    negv = jnp.full((16,), NEG, jnp.float32)

    def per_b(bi, _):
        qb = bi * CB
        pltpu.sync_copy(v_hbm.at[pl.ds(base + qb, CB)], row_v)
        pltpu.sync_copy(d_hbm.at[pl.ds(base + qb, CB)], drow_v)
        for l0 in range(CB):
            for t in range(OUTW // 16):
                vout_v[l0, pl.ds(t * 16, 16)] = negv

        def per_q(l, _1):
            tsp = plsc.load_gather(tau_v, [jnp.full((16,), qb + l, jnp.int32)])
            lsplat = jnp.full((16,), l, jnp.int32)

            def per_g(g, cur):
                v = row_v[l, pl.ds(g * 16, 16)]
                d = drow_v[l, pl.ds(g * 16, 16)]
                msk = v >= tsp
                mi = msk.astype(jnp.int32)
                excl = plsc.cumsum(mi) - mi
                pos = jnp.minimum(excl + cur, OUTW - 1)
                plsc.store_scatter(vout_v, [lsplat, pos], v, mask=msk)
                plsc.store_scatter(dout_v, [lsplat, pos], d, mask=msk)
                return cur + jnp.sum(mi)

            lax.fori_loop(0, w // 16, per_g, 0)
            return 0

        lax.fori_loop(0, CB, per_q, 0)
        pltpu.sync_copy(vout_v.at[:, pl.ds(0, 128)],
                        vo_hbm.at[pl.ds(base + qb, CB)])
        pltpu.sync_copy(dout_v.at[:, pl.ds(0, 128)],
                        do_hbm.at[pl.ds(base + qb, CB)])
        return 0

    lax.fori_loop(0, SUBQ // CB, per_b, 0)


def _cpair(vals, docs, tau_flat, w):
    mesh = plsc.VectorSubcoreMesh(core_axis_name="c", subcore_axis_name="s")
    f = functools.partial(
        pl.kernel,
        out_type=[
            jax.ShapeDtypeStruct((NQ, 128), jnp.float32),
            jax.ShapeDtypeStruct((NQ, 128), jnp.int32),
        ],
        mesh=mesh,
        scratch_types=[
            pltpu.VMEM((CB, w), jnp.float32),
            pltpu.VMEM((CB, w), jnp.int32),
            pltpu.VMEM((SUBQ,), jnp.float32),
            pltpu.VMEM((CB, OUTW), jnp.float32),
            pltpu.VMEM((CB, OUTW), jnp.int32),
        ],
        compiler_params=pltpu.CompilerParams(needs_layout_passes=False),
    )(functools.partial(_cpair_body, w))
    return f(vals, docs, tau_flat)


def _sort_body(vc_ref, dc_ref, vals_ref, docs_ref):
    vals_ref[...] = jnp.zeros_like(vals_ref)
    docs_ref[...] = jnp.zeros_like(docs_ref)
    d = dc_ref[...]

    def body(i, b):
        mx = jnp.max(b, axis=1)
        eq = b == mx[:, None]
        docv = jnp.min(jnp.where(eq, d, BIGI), axis=1)
        eq2 = eq & (d == docv[:, None])
        vals_ref[pl.ds(i, 1), :] = mx.reshape(1, NQ)
        docs_ref[pl.ds(i, 1), :] = docv.reshape(1, NQ)
        return jnp.where(eq2, NEG, b)

    lax.fori_loop(0, TOPK, body, vc_ref[...])


def _sort100(vc, dc):
    return pl.pallas_call(
        _sort_body,
        out_shape=[
            jax.ShapeDtypeStruct((128, NQ), jnp.float32),
            jax.ShapeDtypeStruct((128, NQ), jnp.int32),
        ],
    )(vc, dc)


def _final(cand, cand_doc):
    tau = _tau(cand, NFIN)
    vc, dc = _cpair(cand, cand_doc, tau.reshape(NQ), NFIN)
    return _sort100(vc, dc)


def kernel(queries, keys, k):
    logits, m3 = _matmul(queries, keys)
    m = m3.transpose(1, 0, 2).reshape(NQ, NCHUNK)

    sel_rows = _select(m, NCHUNK)
    sel_t = sel_rows[:, :TOPK]

    qid = jnp.arange(NQ, dtype=jnp.int32)[:, None]
    idx_flat = (qid * NCHUNK + sel_t).reshape(ROWS_TOTAL)
    table = logits.reshape(NQ * NCHUNK, CHUNK)
    cand = _gather(table, idx_flat, CHUNK).reshape(NQ, NSEL)

    m2, tau2 = _submax(cand)
    sel2_rows = _compact(m2, tau2.reshape(NQ), NSUB)

    cand2, cand_doc = _subgather(cand, sel2_rows, sel_rows)

    vals, docs = _final(cand2, cand_doc)
    values = vals[:TOPK].T
    indices = docs[:TOPK].T
    return values, indices


# speedup vs baseline: 1.0165x; 1.0165x over previous
import functools

import jax
import jax.numpy as jnp
from jax import lax
from jax.experimental import pallas as pl
from jax.experimental.pallas import tpu as pltpu
from jax.experimental.pallas import tpu_sc as plsc

NQ = 1024
NKEYS = 100000
CHUNK = 128
NCHUNK = 784
NPAD = NCHUNK * CHUNK
KB = 2048
TOPK = 100
NSEL = TOPK * CHUNK
NEG = -3.0e38
BIGI = 2**30

def _mm_body(q_ref, k_ref, out_ref, m_ref):
    j = pl.program_id(0)
    s = lax.dot_general(q_ref[...], k_ref[...],
                        dimension_numbers=(((1,), (1,)), ((), ())),
                        preferred_element_type=jnp.float32)
    col = j * KB + lax.broadcasted_iota(jnp.int32, s.shape, 1)
    s = jnp.where(col < NKEYS, s, NEG)
    out_ref[...] = s
    m = jnp.max(s.reshape(NQ, KB // CHUNK, CHUNK), axis=2)
    m_ref[...] = m.reshape(1, NQ, KB // CHUNK)


def _matmul(queries, keys_p):
    return pl.pallas_call(
        _mm_body,
        grid=(NPAD // KB,),
        in_specs=[
            pl.BlockSpec((NQ, 128), lambda j: (0, 0)),
            pl.BlockSpec((KB, 128), lambda j: (j, 0)),
        ],
        out_specs=[
            pl.BlockSpec((NQ, KB), lambda j: (0, j)),
            pl.BlockSpec((1, NQ, KB // CHUNK), lambda j: (j, 0, 0)),
        ],
        out_shape=[
            jax.ShapeDtypeStruct((NQ, NPAD), jnp.float32),
            jax.ShapeDtypeStruct((NPAD // KB, NQ, KB // CHUNK), jnp.float32),
        ],
    )(queries, keys_p)


MASK31 = 0x7FFFFFFF


def _tau_body(w, m_ref, tau_ref, sbuf):
    ib = lax.bitcast_convert_type(m_ref[...], jnp.int32)
    sbuf[...] = jnp.where(ib >= 0, ib, ib ^ MASK31)

    minint = jnp.int32(-2**31)

    def body(i, prefix_v):
        trial_v = prefix_v | lax.shift_left(jnp.int32(1), 31 - i)
        trial_s = trial_v ^ minint
        cnt = jnp.sum((sbuf[...] >= trial_s[:, None]).astype(jnp.int32),
                      axis=1)
        return jnp.where(cnt >= TOPK, trial_v, prefix_v)

    pfx_v = lax.fori_loop(0, 32, body, jnp.zeros((NQ,), jnp.int32))
    pfx = pfx_v ^ minint
    tb = jnp.where(pfx >= 0, pfx, pfx ^ MASK31)
    tau_ref[...] = lax.bitcast_convert_type(tb, jnp.float32).reshape(NQ, 1)


def _tau(m, w):
    return pl.pallas_call(
        functools.partial(_tau_body, w),
        out_shape=jax.ShapeDtypeStruct((NQ, 1), jnp.float32),
        scratch_shapes=[pltpu.VMEM((NQ, w), jnp.int32)],
    )(m)


CB = 8
OUTW = 144


def _compact_body(w, m_hbm, tau_hbm, ids_hbm, row_v, tau_v, out_v):
    wid = lax.axis_index("s") * 2 + lax.axis_index("c")
    base = wid * SUBQ
    pltpu.sync_copy(tau_hbm.at[pl.ds(base, SUBQ)], tau_v)
    iota16 = lax.broadcasted_iota(jnp.int32, (16,), 0)

    def per_b(bi, _):
        qb = bi * CB
        pltpu.sync_copy(m_hbm.at[pl.ds(base + qb, CB)], row_v)

        def per_q(l, _1):
            tsp = plsc.load_gather(tau_v, [jnp.full((16,), qb + l, jnp.int32)])

            def per_g(g, cur):
                v = row_v[l, pl.ds(g * 16, 16)]
                msk = v >= tsp
                mi = msk.astype(jnp.int32)
                excl = plsc.cumsum(mi) - mi
                pos = jnp.minimum(excl + cur, OUTW - 1)
                ids = plsc.bitcast(g * 16 + iota16, jnp.float32)
                plsc.store_scatter(out_v, [jnp.full((16,), l, jnp.int32),
                                           pos], ids, mask=msk)
                return cur + jnp.sum(mi)

            lax.fori_loop(0, w // 16, per_g, 0)
            return 0

        lax.fori_loop(0, CB, per_q, 0)
        pltpu.sync_copy(out_v.at[:, pl.ds(0, 128)],
                        ids_hbm.at[pl.ds(base + qb, CB)])
        return 0

    lax.fori_loop(0, SUBQ // CB, per_b, 0)


def _compact(m, tau_flat, w):
    mesh = plsc.VectorSubcoreMesh(core_axis_name="c", subcore_axis_name="s")
    f = functools.partial(
        pl.kernel,
        out_type=jax.ShapeDtypeStruct((NQ, 128), jnp.float32),
        mesh=mesh,
        scratch_types=[
            pltpu.VMEM((CB, w), jnp.float32),
            pltpu.VMEM((SUBQ,), jnp.float32),
            pltpu.VMEM((CB, OUTW), jnp.float32),
        ],
        compiler_params=pltpu.CompilerParams(needs_layout_passes=False),
    )(functools.partial(_compact_body, w))
    return lax.bitcast_convert_type(f(m, tau_flat), jnp.int32)


def _select(m, w):
    tau = _tau(m, w)
    return _compact(m, tau.reshape(NQ), w)


NW = 32
ROWS_TOTAL = NQ * TOPK
ROWS_PER_W = ROWS_TOTAL // NW
WIN = 128
NWIN = ROWS_PER_W // WIN


def _gather_body(width, table_hbm, idx_hbm, out_hbm, idx_v, rows_v, sem):
    wid = lax.axis_index("s") * 2 + lax.axis_index("c")
    base0 = wid * ROWS_PER_W

    def win(w, _):
        base = base0 + w * WIN
        pltpu.sync_copy(idx_hbm.at[pl.ds(base, WIN)], idx_v)
        pltpu.async_copy(table_hbm.at[idx_v], rows_v, sem).wait()
        pltpu.sync_copy(rows_v, out_hbm.at[pl.ds(base, WIN)])
        return 0

    lax.fori_loop(0, NWIN, win, 0)


def _gather(table, idx_flat, width):
    mesh = plsc.VectorSubcoreMesh(core_axis_name="c", subcore_axis_name="s")
    f = functools.partial(
        pl.kernel,
        out_type=jax.ShapeDtypeStruct((ROWS_TOTAL, width), jnp.float32),
        mesh=mesh,
        scratch_types=[
            pltpu.VMEM((WIN,), jnp.int32),
            pltpu.VMEM((WIN, width), jnp.float32),
            pltpu.SemaphoreType.DMA,
        ],
    )(functools.partial(_gather_body, width))
    return f(table, idx_flat)


SUBQ = NQ // NW


SB = 4


def _subgather_body(cand_hbm, sel2_hbm, selrow_hbm, vals_hbm, docs_hbm,
                    row_v, ids_v, selrow_v, out_v, doc_v):
    wid = lax.axis_index("s") * 2 + lax.axis_index("c")
    base = wid * SUBQ
    iota16 = lax.broadcasted_iota(jnp.int32, (16,), 0)

    def per_b(bi, _):
        qb = base + bi * SB
        pltpu.sync_copy(cand_hbm.at[pl.ds(qb, SB)], row_v)
        pltpu.sync_copy(sel2_hbm.at[pl.ds(qb, SB)], ids_v)
        pltpu.sync_copy(selrow_hbm.at[pl.ds(qb, SB)], selrow_v)

        def per_q(l, _1):
            lsplat = jnp.full((16,), l, jnp.int32)

            def per_j(j, _2):
                jsplat = jnp.full((16,), j, dtype=jnp.int32)
                sid_f = plsc.load_gather(ids_v, [lsplat, jsplat])
                sid = plsc.bitcast(sid_f, jnp.int32)
                pos = sid + NSUB * iota16
                out_v[l, pl.ds(j * SUB, SUB)] = (
                    plsc.load_gather(row_v, [lsplat, pos]))
                crank = lax.shift_right_logical(pos, 7)
                cid = plsc.bitcast(
                    plsc.load_gather(selrow_v, [lsplat, crank]), jnp.int32)
                doc_v[l, pl.ds(j * SUB, SUB)] = (cid * CHUNK
                                                 + (pos & (CHUNK - 1)))
                return 0

            lax.fori_loop(0, TOPK, per_j, 0)
            return 0

        lax.fori_loop(0, SB, per_q, 0)
        pltpu.sync_copy(out_v, vals_hbm.at[pl.ds(qb, SB)])
        pltpu.sync_copy(doc_v, docs_hbm.at[pl.ds(qb, SB)])
        return 0

    lax.fori_loop(0, SUBQ // SB, per_b, 0)


def _subgather(cand, sel2_rows, sel_rows):
    mesh = plsc.VectorSubcoreMesh(core_axis_name="c", subcore_axis_name="s")
    f = functools.partial(
        pl.kernel,
        out_type=[
            jax.ShapeDtypeStruct((NQ, TOPK * SUB), jnp.float32),
            jax.ShapeDtypeStruct((NQ, TOPK * SUB), jnp.int32),
        ],
        mesh=mesh,
        scratch_types=[
            pltpu.VMEM((SB, NSEL), jnp.float32),
            pltpu.VMEM((SB, 128), jnp.float32),
            pltpu.VMEM((SB, 128), jnp.float32),
            pltpu.VMEM((SB, TOPK * SUB), jnp.float32),
            pltpu.VMEM((SB, TOPK * SUB), jnp.int32),
        ],
        compiler_params=pltpu.CompilerParams(needs_layout_passes=False),
    )(_subgather_body)
    return f(cand, lax.bitcast_convert_type(sel2_rows, jnp.float32),
             lax.bitcast_convert_type(sel_rows, jnp.float32))


SUB = 16
NSUB = NSEL // SUB
QBA = 256


def _submax_body(c_ref, m2_ref):
    u = c_ref[...]
    h = NSEL
    while h > NSUB:
        h //= 2
        u = jnp.maximum(u[:, :h], u[:, h:])
    m2_ref[...] = u


def _submax(cand):
    return pl.pallas_call(
        _submax_body,
        grid=(NQ // QBA,),
        in_specs=[pl.BlockSpec((QBA, NSEL), lambda qi: (qi, 0))],
        out_specs=pl.BlockSpec((QBA, NSUB), lambda qi: (qi, 0)),
        out_shape=jax.ShapeDtypeStruct((NQ, NSUB), jnp.float32),
    )(cand)


NFIN = TOPK * SUB


def _cpair_body(w, v_hbm, d_hbm, tau_hbm, vo_hbm, do_hbm,
                row_v, drow_v, tau_v, vout_v, dout_v):
    wid = lax.axis_index("s") * 2 + lax.axis_index("c")
    base = wid * SUBQ
    pltpu.sync_copy(tau_hbm.at[pl.ds(base, SUBQ)], tau_v)
    negv = jnp.full((16,), NEG, jnp.float32)

    def per_b(bi, _):
        qb = bi * CB
        pltpu.sync_copy(v_hbm.at[pl.ds(base + qb, CB)], row_v)
        pltpu.sync_copy(d_hbm.at[pl.ds(base + qb, CB)], drow_v)
        for l0 in range(CB):
            for t in range(OUTW // 16):
                vout_v[l0, pl.ds(t * 16, 16)] = negv

        def per_q(l, _1):
            tsp = plsc.load_gather(tau_v, [jnp.full((16,), qb + l, jnp.int32)])
            lsplat = jnp.full((16,), l, jnp.int32)

            def per_g(g, cur):
                v = row_v[l, pl.ds(g * 16, 16)]
                d = drow_v[l, pl.ds(g * 16, 16)]
                msk = v >= tsp
                mi = msk.astype(jnp.int32)
                excl = plsc.cumsum(mi) - mi
                pos = jnp.minimum(excl + cur, OUTW - 1)
                plsc.store_scatter(vout_v, [lsplat, pos], v, mask=msk)
                plsc.store_scatter(dout_v, [lsplat, pos], d, mask=msk)
                return cur + jnp.sum(mi)

            lax.fori_loop(0, w // 16, per_g, 0)
            return 0

        lax.fori_loop(0, CB, per_q, 0)
        pltpu.sync_copy(vout_v.at[:, pl.ds(0, 128)],
                        vo_hbm.at[pl.ds(base + qb, CB)])
        pltpu.sync_copy(dout_v.at[:, pl.ds(0, 128)],
                        do_hbm.at[pl.ds(base + qb, CB)])
        return 0

    lax.fori_loop(0, SUBQ // CB, per_b, 0)


def _cpair(vals, docs, tau_flat, w):
    mesh = plsc.VectorSubcoreMesh(core_axis_name="c", subcore_axis_name="s")
    f = functools.partial(
        pl.kernel,
        out_type=[
            jax.ShapeDtypeStruct((NQ, 128), jnp.float32),
            jax.ShapeDtypeStruct((NQ, 128), jnp.int32),
        ],
        mesh=mesh,
        scratch_types=[
            pltpu.VMEM((CB, w), jnp.float32),
            pltpu.VMEM((CB, w), jnp.int32),
            pltpu.VMEM((SUBQ,), jnp.float32),
            pltpu.VMEM((CB, OUTW), jnp.float32),
            pltpu.VMEM((CB, OUTW), jnp.int32),
        ],
        compiler_params=pltpu.CompilerParams(needs_layout_passes=False),
    )(functools.partial(_cpair_body, w))
    return f(vals, docs, tau_flat)


def _sort_body(vc_ref, dc_ref, vals_ref, docs_ref):
    vals_ref[...] = jnp.zeros_like(vals_ref)
    docs_ref[...] = jnp.zeros_like(docs_ref)
    rows128 = lax.broadcasted_iota(jnp.int32, (128, NQ), 0)
    d = dc_ref[...]

    def body(i, b):
        mx = jnp.max(b, axis=1)
        eq = b == mx[:, None]
        docv = jnp.min(jnp.where(eq, d, BIGI), axis=1)
        eq2 = eq & (d == docv[:, None])
        hit = rows128 == i
        vals_ref[...] = jnp.where(hit, mx.reshape(1, NQ), vals_ref[...])
        docs_ref[...] = jnp.where(hit, docv.reshape(1, NQ), docs_ref[...])
        return jnp.where(eq2, NEG, b)

    lax.fori_loop(0, TOPK, body, vc_ref[...])


def _sort100(vc, dc):
    return pl.pallas_call(
        _sort_body,
        out_shape=[
            jax.ShapeDtypeStruct((128, NQ), jnp.float32),
            jax.ShapeDtypeStruct((128, NQ), jnp.int32),
        ],
    )(vc, dc)


def _final(cand, cand_doc):
    tau = _tau(cand, NFIN)
    vc, dc = _cpair(cand, cand_doc, tau.reshape(NQ), NFIN)
    return _sort100(vc, dc)


def kernel(queries, keys, k):
    n = keys.shape[0]
    keys_p = jnp.pad(keys, ((0, NPAD - n), (0, 0)))
    logits, m3 = _matmul(queries, keys_p)
    m = m3.transpose(1, 0, 2).reshape(NQ, NCHUNK)

    sel_rows = _select(m, NCHUNK)
    sel_t = sel_rows[:, :TOPK]

    qid = jnp.arange(NQ, dtype=jnp.int32)[:, None]
    idx_flat = (qid * NCHUNK + sel_t).reshape(ROWS_TOTAL)
    table = logits.reshape(NQ * NCHUNK, CHUNK)
    cand = _gather(table, idx_flat, CHUNK).reshape(NQ, NSEL)

    m2 = _submax(cand)
    sel2_rows = _select(m2, NSUB)

    cand2, cand_doc = _subgather(cand, sel2_rows, sel_rows)

    vals, docs = _final(cand2, cand_doc)
    values = vals[:TOPK].T
    indices = docs[:TOPK].T
    return values, indices
